# Initial kernel scaffold; baseline (speedup 1.0000x reference)
import jax
import jax.numpy as jnp
from jax.experimental import pallas as pl


def kernel(attr, pot, edge_index, edge_d, Wp, bp, Wv0, Ww0, Wu0, Wa0, gate0, Wv1, Ww1, Wu1, Wa1, gate1):
    raise NotImplementedError("write your pallas kernel here")


# speedup vs baseline: 4.8041x; 4.8041x over previous
import functools

import jax
import jax.numpy as jnp
from jax import lax
from jax.experimental import pallas as pl
from jax.experimental.pallas import tpu as pltpu
from jax.experimental.pallas import tpu_sc as plsc

N = 50000
E = 800000
IN_DIM = 128
POT_DIM = 16
H = 32
W = 40

NC = 2
NS = 16
L = 16

NH = 25600
E_PAD = 819200
ET = E_PAD // NS
C = 128
G = C // L
ROWS_T = NH // NS
RCHUNK = 100
BN = 400
BE = 6400


def _pre0_body(attr, pot, wpt, bpv, wwt, wut, was, vi, z, zi, zab):
    vi[...] = jnp.tanh(jnp.dot(pot[...], wpt[...],
                               preferred_element_type=jnp.float32) + bpv[...][None, :])
    zb = jnp.dot(attr[...], wwt[...], preferred_element_type=jnp.float32)
    z[...] = zb
    zi[...] = jnp.dot(attr[...], wut[...], preferred_element_type=jnp.float32)
    zab[...] = jnp.dot(zb, was[...], preferred_element_type=jnp.float32)


def _ta_body(ed, wes, tab):
    tab[...] = jnp.dot(ed[...], wes[...], preferred_element_type=jnp.float32)


def _combine(acc, zi, vi, gate):
    accs = acc[...][0]
    dcol = accs[:, H:H + 1]
    znb = accs[:, :H] / jnp.maximum(dcol, 1e-16)
    h = znb + gate[...][None, :] * vi[...] * zi[...]
    return jnp.where(dcol > 0, h, 0.0)


def _mid_body(acc, zi, vi, gatev, wwt, wut, wab, z, zi1, zab):
    h = _combine(acc, zi, vi, gatev)
    z[...] = jnp.dot(h, wwt[...], preferred_element_type=jnp.float32)
    zi1[...] = jnp.dot(h, wut[...], preferred_element_type=jnp.float32)
    zab[...] = jnp.dot(h, wab[...], preferred_element_type=jnp.float32)


def _final_body(acc, zi, vi, gatev, out):
    out[...] = _combine(acc, zi, vi, gatev)


def _node_spec():
    return pl.BlockSpec((BN, H), lambda i: (i, 0))


def _full(shape):
    return pl.BlockSpec(shape, lambda i: tuple(0 for _ in shape))


def _pre0(attr, pot, wpt, bpv, wwt, wut, was):
    f = jnp.float32
    return pl.pallas_call(
        _pre0_body,
        grid=(N // BN,),
        in_specs=[
            pl.BlockSpec((BN, IN_DIM), lambda i: (i, 0)),
            pl.BlockSpec((BN, POT_DIM), lambda i: (i, 0)),
            _full((POT_DIM, H)), _full((H,)), _full((IN_DIM, H)),
            _full((IN_DIM, H)), _full((H, 16)),
        ],
        out_specs=[_node_spec(), _node_spec(), _node_spec(),
                   pl.BlockSpec((BN, 16), lambda i: (i, 0))],
        out_shape=[jax.ShapeDtypeStruct((N, H), f)] * 3
        + [jax.ShapeDtypeStruct((N, 16), f)],
    )(attr, pot, wpt, bpv, wwt, wut, was)


def _ta(edge_d, wes):
    return pl.pallas_call(
        _ta_body,
        grid=(E // BE,),
        in_specs=[pl.BlockSpec((BE, H), lambda i: (i, 0)), _full((H, 8))],
        out_specs=pl.BlockSpec((BE, 8), lambda i: (i, 0)),
        out_shape=jax.ShapeDtypeStruct((E, 8), jnp.float32),
    )(edge_d, wes)


def _acc_spec():
    return pl.BlockSpec((1, BN, W), lambda i: (i // (NH // BN), i % (NH // BN), 0))


def _mid(acc, zi, vi, gate, wwt, wut, wab):
    f = jnp.float32
    return pl.pallas_call(
        _mid_body,
        grid=(N // BN,),
        in_specs=[_acc_spec(), _node_spec(), _node_spec(), _full((H,)),
                  _full((H, H)), _full((H, H)), _full((H, 16))],
        out_specs=[_node_spec(), _node_spec(),
                   pl.BlockSpec((BN, 16), lambda i: (i, 0))],
        out_shape=[jax.ShapeDtypeStruct((N, H), f)] * 2
        + [jax.ShapeDtypeStruct((N, 16), f)],
    )(acc, zi, vi, gate, wwt, wut, wab)


def _final(acc, zi, vi, gate):
    return pl.pallas_call(
        _final_body,
        grid=(N // BN,),
        in_specs=[_acc_spec(), _node_spec(), _node_spec(), _full((H,))],
        out_specs=_node_spec(),
        out_shape=jax.ShapeDtypeStruct((N, H), jnp.float32),
    )(acc, zi, vi, gate)


def _sc_body(col, z_hbm, zab_hbm, tab_hbm, src_hbm, dst_hbm, zrow_hbm,
             acc_out,
             src_v, dst_v, dstloc_v, ta_v, zas_v, zbd_v, zrows, srows,
             acc_sh, sem):
    cid = lax.axis_index("c")
    sid = lax.axis_index("s")
    iota16 = lax.iota(jnp.int32, L)

    pltpu.sync_copy(zrow_hbm, srows)
    row0 = sid * ROWS_T
    for i in range(ROWS_T // RCHUNK):
        pltpu.sync_copy(srows.at[pl.ds(0, RCHUNK)],
                        acc_sh.at[pl.ds(row0 + i * RCHUNK, RCHUNK)])
    plsc.subcore_barrier()

    lo = cid * NH
    col_v = jnp.full((L,), col, jnp.int32)
    zacol_v = jnp.zeros((L,), jnp.int32)
    zbcol_v = jnp.full((L,), 1, jnp.int32)
    den_v = jnp.full((L,), H, jnp.int32)

    def chunk(t, carry):
        base = sid * ET + t * C
        pltpu.sync_copy(src_hbm.at[pl.ds(base, C)], src_v)
        pltpu.sync_copy(dst_hbm.at[pl.ds(base, C)], dst_v)
        pltpu.sync_copy(tab_hbm.at[pl.ds(base, C)], ta_v)
        pltpu.async_copy(z_hbm.at[src_v], zrows, sem).wait()
        pltpu.async_copy(zab_hbm.at[src_v], zas_v, sem).wait()
        pltpu.async_copy(zab_hbm.at[dst_v], zbd_v, sem).wait()
        for g in range(G):
            sl = pl.ds(g * L, L)
            rid = g * L + iota16
            zav = plsc.load_gather(zas_v, [rid, zacol_v])
            zbv = plsc.load_gather(zbd_v, [rid, zbcol_v])
            tav = plsc.load_gather(ta_v, [rid, col_v])
            a = zav + zbv + tav
            s = jnp.exp(jnp.where(a > 0, a, 0.01 * a))
            loc = dst_v[sl] - lo
            owned = (loc >= 0) & (loc < NH) & (base + rid < E)
            s = jnp.where(owned, s, 0.0)
            dstloc_v[sl] = jnp.where(owned, loc, 0)
            plsc.store_scatter(srows, [rid, den_v], s)
            for d in range(H):
                dcol = jnp.full((L,), d, jnp.int32)
                cvals = plsc.load_gather(zrows, [rid, dcol])
                plsc.store_scatter(srows, [rid, dcol], cvals * s)
        pltpu.sync_copy(srows, acc_sh.at[dstloc_v], add=True)
        return carry

    lax.fori_loop(0, ET // C, chunk, 0)

    plsc.subcore_barrier()
    for i in range(ROWS_T // RCHUNK):
        r = row0 + i * RCHUNK
        pltpu.sync_copy(acc_sh.at[pl.ds(r, RCHUNK)], srows.at[pl.ds(0, RCHUNK)])
        pltpu.sync_copy(srows.at[pl.ds(0, RCHUNK)],
                        acc_out.at[cid, pl.ds(r, RCHUNK)])


def _sc_edge_kernel(col):
    f = jnp.float32
    mesh = plsc.VectorSubcoreMesh(core_axis_name="c", subcore_axis_name="s",
                                  num_cores=NC, num_subcores=NS)
    return pl.kernel(
        functools.partial(_sc_body, col),
        out_type=jax.ShapeDtypeStruct((NC, NH, W), f),
        mesh=mesh,
        compiler_params=pltpu.CompilerParams(needs_layout_passes=False,
                                             use_tc_tiling_on_sc=False),
        scratch_types=[
            pltpu.VMEM((C,), jnp.int32),
            pltpu.VMEM((C,), jnp.int32),
            pltpu.VMEM((C,), jnp.int32),
            pltpu.VMEM((C, 8), f),
            pltpu.VMEM((C, 16), f),
            pltpu.VMEM((C, 16), f),
            pltpu.VMEM((C, H), f),
            pltpu.VMEM((C, W), f),
            pltpu.VMEM_SHARED((NH, W), f),
            pltpu.SemaphoreType.DMA,
        ],
    )


def kernel(attr, pot, edge_index, edge_d, Wp, bp,
           Wv0, Ww0, Wu0, Wa0, gate0,
           Wv1, Ww1, Wu1, Wa1, gate1):
    f = jnp.float32
    src = edge_index[0]
    dst = edge_index[1]

    was0 = jnp.zeros((H, 16), f).at[:, 0].set(Wa0[0, :H]).at[:, 1].set(Wa0[0, H:2 * H])
    was1 = jnp.zeros((H, 16), f).at[:, 0].set(Wa1[0, :H]).at[:, 1].set(Wa1[0, H:2 * H])
    wes = (jnp.zeros((H, 8), f)
           .at[:, 0].set(Wv0.T @ Wa0[0, 2 * H:])
           .at[:, 1].set(Wv1.T @ Wa1[0, 2 * H:]))
    wab1 = Ww1.T @ was1

    pad = E_PAD - E
    src_p = jnp.concatenate([src, jnp.zeros((pad,), jnp.int32)])
    dst_p = jnp.concatenate([dst, jnp.zeros((pad,), jnp.int32)])

    vi, z0, zi0, zab0 = _pre0(attr, pot, Wp.T, bp, Ww0.T, Wu0.T, was0)
    tab = _ta(edge_d, wes)
    tab_p = jnp.concatenate([tab, jnp.zeros((pad, 8), f)])
    zrow0s = jnp.zeros((C, W), f)

    acc0 = _sc_edge_kernel(0)(z0, zab0, tab_p, src_p, dst_p, zrow0s)
    z1, zi1, zab1 = _mid(acc0, zi0, vi, gate0, Ww1.T, Wu1.T, wab1)
    acc1 = _sc_edge_kernel(1)(z1, zab1, tab_p, src_p, dst_p, zrow0s)
    return _final(acc1, zi1, vi, gate1)


# speedup vs baseline: 6.3483x; 1.3214x over previous
import functools

import jax
import jax.numpy as jnp
from jax import lax
from jax.experimental import pallas as pl
from jax.experimental.pallas import tpu as pltpu
from jax.experimental.pallas import tpu_sc as plsc

N = 50000
E = 800000
IN_DIM = 128
POT_DIM = 16
H = 32
W = 40

NC = 2
NS = 16
L = 16

NH = 25600
E_PAD = 819200
ET = E_PAD // NS
C = 128
G = C // L
ROWS_T = NH // NS
RCHUNK = 100
BN = 400
BE = 6400


def _pre0_body(attr, pot, wpt, bpv, wwt, wut, was, vi, z, zi, zab):
    vi[...] = jnp.tanh(jnp.dot(pot[...], wpt[...],
                               preferred_element_type=jnp.float32) + bpv[...][None, :])
    zb = jnp.dot(attr[...], wwt[...], preferred_element_type=jnp.float32)
    z[...] = zb
    zi[...] = jnp.dot(attr[...], wut[...], preferred_element_type=jnp.float32)
    zab[...] = jnp.dot(zb, was[...], preferred_element_type=jnp.float32)


def _ta_body(ed, wes, tab):
    tab[...] = jnp.dot(ed[...], wes[...], preferred_element_type=jnp.float32)


def _combine(acc, zi, vi, gate):
    accs = acc[...][0]
    dcol = accs[:, H:H + 1]
    znb = accs[:, :H] / jnp.maximum(dcol, 1e-16)
    h = znb + gate[...][None, :] * vi[...] * zi[...]
    return jnp.where(dcol > 0, h, 0.0)


def _mid_body(acc, zi, vi, gatev, wwt, wut, wab, z, zi1, zab):
    h = _combine(acc, zi, vi, gatev)
    z[...] = jnp.dot(h, wwt[...], preferred_element_type=jnp.float32)
    zi1[...] = jnp.dot(h, wut[...], preferred_element_type=jnp.float32)
    zab[...] = jnp.dot(h, wab[...], preferred_element_type=jnp.float32)


def _final_body(acc, zi, vi, gatev, out):
    out[...] = _combine(acc, zi, vi, gatev)


def _node_spec():
    return pl.BlockSpec((BN, H), lambda i: (i, 0))


def _full(shape):
    return pl.BlockSpec(shape, lambda i: tuple(0 for _ in shape))


def _pre0(attr, pot, wpt, bpv, wwt, wut, was):
    f = jnp.float32
    return pl.pallas_call(
        _pre0_body,
        grid=(N // BN,),
        in_specs=[
            pl.BlockSpec((BN, IN_DIM), lambda i: (i, 0)),
            pl.BlockSpec((BN, POT_DIM), lambda i: (i, 0)),
            _full((POT_DIM, H)), _full((H,)), _full((IN_DIM, H)),
            _full((IN_DIM, H)), _full((H, 16)),
        ],
        out_specs=[_node_spec(), _node_spec(), _node_spec(),
                   pl.BlockSpec((BN, 16), lambda i: (i, 0))],
        out_shape=[jax.ShapeDtypeStruct((N, H), f)] * 3
        + [jax.ShapeDtypeStruct((N, 16), f)],
    )(attr, pot, wpt, bpv, wwt, wut, was)


def _ta(edge_d, wes):
    return pl.pallas_call(
        _ta_body,
        grid=(E // BE,),
        in_specs=[pl.BlockSpec((BE, H), lambda i: (i, 0)), _full((H, 8))],
        out_specs=pl.BlockSpec((BE, 8), lambda i: (i, 0)),
        out_shape=jax.ShapeDtypeStruct((E, 8), jnp.float32),
    )(edge_d, wes)


def _acc_spec():
    return pl.BlockSpec((1, BN, W), lambda i: (i // (NH // BN), i % (NH // BN), 0))


def _mid(acc, zi, vi, gate, wwt, wut, wab):
    f = jnp.float32
    return pl.pallas_call(
        _mid_body,
        grid=(N // BN,),
        in_specs=[_acc_spec(), _node_spec(), _node_spec(), _full((H,)),
                  _full((H, H)), _full((H, H)), _full((H, 16))],
        out_specs=[_node_spec(), _node_spec(),
                   pl.BlockSpec((BN, 16), lambda i: (i, 0))],
        out_shape=[jax.ShapeDtypeStruct((N, H), f)] * 2
        + [jax.ShapeDtypeStruct((N, 16), f)],
    )(acc, zi, vi, gate, wwt, wut, wab)


def _final(acc, zi, vi, gate):
    return pl.pallas_call(
        _final_body,
        grid=(N // BN,),
        in_specs=[_acc_spec(), _node_spec(), _node_spec(), _full((H,))],
        out_specs=_node_spec(),
        out_shape=jax.ShapeDtypeStruct((N, H), jnp.float32),
    )(acc, zi, vi, gate)


def _sc_body(col, z_hbm, zab_hbm, tab_hbm, src_hbm, dst_hbm, zrow_hbm,
             acc_out,
             src_a, dst_a, ta_a, src_b, dst_b, ta_b,
             dstloc_v, zas_v, zbd_v, zrows, srows,
             acc_sh, sem_a, sem_b, sem_g):
    cid = lax.axis_index("c")
    sid = lax.axis_index("s")
    iota16 = lax.iota(jnp.int32, L)

    pltpu.sync_copy(zrow_hbm, srows)
    row0 = sid * ROWS_T
    for i in range(ROWS_T // RCHUNK):
        pltpu.sync_copy(srows.at[pl.ds(0, RCHUNK)],
                        acc_sh.at[pl.ds(row0 + i * RCHUNK, RCHUNK)])
    plsc.subcore_barrier()

    lo = cid * NH
    col_v = jnp.full((L,), col, jnp.int32)
    zacol_v = jnp.zeros((L,), jnp.int32)
    zbcol_v = jnp.full((L,), 1, jnp.int32)
    den_v = jnp.full((L,), H, jnp.int32)

    def lin_issue(t, sv, dv, tv, sem):
        base = jnp.minimum(sid * ET + t * C, E_PAD - C)
        pltpu.async_copy(src_hbm.at[pl.ds(base, C)], sv, sem)
        pltpu.async_copy(dst_hbm.at[pl.ds(base, C)], dv, sem)
        pltpu.async_copy(tab_hbm.at[pl.ds(base, C)], tv, sem)

    def lin_drain(sv, dv, tv, sem):
        pltpu.make_async_copy(src_hbm.at[pl.ds(0, C)], sv, sem).wait()
        pltpu.make_async_copy(dst_hbm.at[pl.ds(0, C)], dv, sem).wait()
        pltpu.make_async_copy(tab_hbm.at[pl.ds(0, C)], tv, sem).wait()

    def process(t, sv, dv, tv):
        base = sid * ET + t * C
        g1 = pltpu.async_copy(z_hbm.at[sv], zrows, sem_g)
        g2 = pltpu.async_copy(zab_hbm.at[sv], zas_v, sem_g)
        g3 = pltpu.async_copy(zab_hbm.at[dv], zbd_v, sem_g)
        g1.wait()
        g2.wait()
        g3.wait()
        for g in range(G):
            sl = pl.ds(g * L, L)
            rid = g * L + iota16
            zav = plsc.load_gather(zas_v, [rid, zacol_v])
            zbv = plsc.load_gather(zbd_v, [rid, zbcol_v])
            tav = plsc.load_gather(tv, [rid, col_v])
            a = zav + zbv + tav
            s = jnp.exp(jnp.where(a > 0, a, 0.01 * a))
            loc = dv[sl] - lo
            owned = (loc >= 0) & (loc < NH) & (base + rid < E)
            s = jnp.where(owned, s, 0.0)
            dstloc_v[sl] = jnp.where(owned, loc, 0)
            plsc.store_scatter(srows, [rid, den_v], s)
            for d in range(H):
                dcol = jnp.full((L,), d, jnp.int32)
                cvals = plsc.load_gather(zrows, [rid, dcol])
                plsc.store_scatter(srows, [rid, dcol], cvals * s)
        pltpu.sync_copy(srows, acc_sh.at[dstloc_v], add=True)

    lin_issue(0, src_a, dst_a, ta_a, sem_a)

    def outer(i, carry):
        t0 = 2 * i
        lin_issue(t0 + 1, src_b, dst_b, ta_b, sem_b)
        lin_drain(src_a, dst_a, ta_a, sem_a)
        process(t0, src_a, dst_a, ta_a)
        lin_issue(t0 + 2, src_a, dst_a, ta_a, sem_a)
        lin_drain(src_b, dst_b, ta_b, sem_b)
        process(t0 + 1, src_b, dst_b, ta_b)
        return carry

    lax.fori_loop(0, ET // C // 2, outer, 0)
    lin_drain(src_a, dst_a, ta_a, sem_a)

    plsc.subcore_barrier()
    for i in range(ROWS_T // RCHUNK):
        r = row0 + i * RCHUNK
        pltpu.sync_copy(acc_sh.at[pl.ds(r, RCHUNK)], srows.at[pl.ds(0, RCHUNK)])
        pltpu.sync_copy(srows.at[pl.ds(0, RCHUNK)],
                        acc_out.at[cid, pl.ds(r, RCHUNK)])


def _sc_edge_kernel(col):
    f = jnp.float32
    mesh = plsc.VectorSubcoreMesh(core_axis_name="c", subcore_axis_name="s",
                                  num_cores=NC, num_subcores=NS)
    return pl.kernel(
        functools.partial(_sc_body, col),
        out_type=jax.ShapeDtypeStruct((NC, NH, W), f),
        mesh=mesh,
        compiler_params=pltpu.CompilerParams(needs_layout_passes=False,
                                             use_tc_tiling_on_sc=False),
        scratch_types=[
            pltpu.VMEM((C,), jnp.int32),
            pltpu.VMEM((C,), jnp.int32),
            pltpu.VMEM((C, 8), f),
            pltpu.VMEM((C,), jnp.int32),
            pltpu.VMEM((C,), jnp.int32),
            pltpu.VMEM((C, 8), f),
            pltpu.VMEM((C,), jnp.int32),
            pltpu.VMEM((C, 16), f),
            pltpu.VMEM((C, 16), f),
            pltpu.VMEM((C, H), f),
            pltpu.VMEM((C, W), f),
            pltpu.VMEM_SHARED((NH, W), f),
            pltpu.SemaphoreType.DMA,
            pltpu.SemaphoreType.DMA,
            pltpu.SemaphoreType.DMA,
        ],
    )


def kernel(attr, pot, edge_index, edge_d, Wp, bp,
           Wv0, Ww0, Wu0, Wa0, gate0,
           Wv1, Ww1, Wu1, Wa1, gate1):
    f = jnp.float32
    src = edge_index[0]
    dst = edge_index[1]

    was0 = jnp.zeros((H, 16), f).at[:, 0].set(Wa0[0, :H]).at[:, 1].set(Wa0[0, H:2 * H])
    was1 = jnp.zeros((H, 16), f).at[:, 0].set(Wa1[0, :H]).at[:, 1].set(Wa1[0, H:2 * H])
    wes = (jnp.zeros((H, 8), f)
           .at[:, 0].set(Wv0.T @ Wa0[0, 2 * H:])
           .at[:, 1].set(Wv1.T @ Wa1[0, 2 * H:]))
    wab1 = Ww1.T @ was1

    pad = E_PAD - E
    src_p = jnp.concatenate([src, jnp.zeros((pad,), jnp.int32)])
    dst_p = jnp.concatenate([dst, jnp.zeros((pad,), jnp.int32)])

    vi, z0, zi0, zab0 = _pre0(attr, pot, Wp.T, bp, Ww0.T, Wu0.T, was0)
    tab = _ta(edge_d, wes)
    tab_p = jnp.concatenate([tab, jnp.zeros((pad, 8), f)])
    zrow0s = jnp.zeros((C, W), f)

    acc0 = _sc_edge_kernel(0)(z0, zab0, tab_p, src_p, dst_p, zrow0s)
    z1, zi1, zab1 = _mid(acc0, zi0, vi, gate0, Ww1.T, Wu1.T, wab1)
    acc1 = _sc_edge_kernel(1)(z1, zab1, tab_p, src_p, dst_p, zrow0s)
    return _final(acc1, zi1, vi, gate1)
